# baseline (speedup 1.0000x reference)
import functools

import jax
import jax.numpy as jnp
from jax import lax
from jax.experimental import pallas as pl
from jax.experimental.pallas import tpu as pltpu
from jax.experimental.pallas import tpu_sc as plsc

B = 8
P = 8
N = 8192
G = 64
V = G ** 3
NPTS = B * P * N
PTS_PER_WORKER = NPTS // 32
C = 1024
NCHUNK = PTS_PER_WORKER // C
NSLOT = 3


def _sc_body(jv_hbm, u16_hbm, tab_hbm, out_hbm, *scratch):
    jv_v = scratch[0:NSLOT]
    u_v = scratch[NSLOT:2 * NSLOT]
    g_v = scratch[2 * NSLOT:3 * NSLOT]
    acc_v = scratch[3 * NSLOT]
    sem_j = scratch[3 * NSLOT + 1:3 * NSLOT + 1 + NSLOT]
    sem_u = scratch[3 * NSLOT + 1 + NSLOT:3 * NSLOT + 1 + 2 * NSLOT]
    sem_g = scratch[3 * NSLOT + 1 + 2 * NSLOT:3 * NSLOT + 1 + 3 * NSLOT]

    wid = lax.axis_index("s") * 2 + lax.axis_index("c")
    base = wid * PTS_PER_WORKER

    def fire_loads(i, s):
        off = base + i * C
        pltpu.async_copy(
            jv_hbm.at[pl.ds(pl.multiple_of(off, C), C)], jv_v[s], sem_j[s])
        pltpu.async_copy(
            u16_hbm.at[pl.ds(pl.multiple_of(off * 16, C * 16), C * 16)],
            u_v[s], sem_u[s])

    def wait_loads(s):
        pltpu.make_async_copy(
            jv_hbm.at[pl.ds(0, C)], jv_v[s], sem_j[s]).wait()
        pltpu.make_async_copy(
            u16_hbm.at[pl.ds(0, C * 16)], u_v[s], sem_u[s]).wait()

    def fire_gather(s):
        pltpu.async_copy(tab_hbm.at[jv_v[s]], g_v[s], sem_g[s])

    def wait_gather(s):
        pltpu.make_async_copy(
            tab_hbm.at[jv_v[s]], g_v[s], sem_g[s]).wait()

    fire_loads(0, 0)
    fire_loads(1, 1)
    wait_loads(0)
    fire_gather(0)

    acc = jnp.zeros((16,), jnp.float32)
    for i in range(NCHUNK):
        s = i % NSLOT
        if i + 1 < NCHUNK:
            sn = (i + 1) % NSLOT
            wait_loads(sn)
            fire_gather(sn)
        if i + 2 < NCHUNK:
            fire_loads(i + 2, (i + 2) % NSLOT)
        wait_gather(s)

        us = u_v[s]
        gs = g_v[s]

        def body_c(r, a):
            return a + us[pl.ds(pl.multiple_of(r * 16, 16), 16)] * \
                gs[r, pl.ds(0, 16)]

        acc = lax.fori_loop(0, C, body_c, acc, unroll=8)

    acc_v[...] = acc
    pltpu.sync_copy(acc_v, out_hbm.at[pl.ds(pl.multiple_of(wid * 16, 16), 16)])


@jax.jit
def _sc_loss(jv, u16, tab):
    mesh = plsc.VectorSubcoreMesh(core_axis_name="c", subcore_axis_name="s")
    f32 = jnp.float32
    i32 = jnp.int32
    scratch = (
        [pltpu.VMEM((C,), i32) for _ in range(NSLOT)]
        + [pltpu.VMEM((C * 16,), f32) for _ in range(NSLOT)]
        + [pltpu.VMEM((C, 16), f32) for _ in range(NSLOT)]
        + [pltpu.VMEM((16,), f32)]
        + [pltpu.SemaphoreType.DMA for _ in range(3 * NSLOT)]
    )
    kern = functools.partial(
        pl.kernel,
        mesh=mesh,
        out_type=jax.ShapeDtypeStruct((32 * 16,), f32),
        scratch_types=scratch,
        compiler_params=pltpu.CompilerParams(use_tc_tiling_on_sc=False),
    )(_sc_body)
    return kern(jv, u16, tab)


def kernel(voxel, points, closest_points, planes):
    f32 = jnp.float32
    m = 1.0 - voxel.reshape(B, V)
    w = (m * m)[..., None]
    wc = closest_points * w
    q = w * jnp.sum(closest_points * closest_points, axis=-1, keepdims=True)
    zeros3 = jnp.zeros((B, V, 3), f32)
    tab = jnp.concatenate([wc, w, q, zeros3], axis=-1)
    tab = tab.reshape(B * V // 2, 16)

    ns = planes[..., 0:3]
    dd = planes[..., 3]
    inv2 = 2.0 / jnp.sum(ns * ns, axis=-1)
    pts = points[:, None, :, :]
    f = (jnp.sum(pts * ns[:, :, None, :], axis=-1) + dd[:, :, None]) \
        * inv2[:, :, None]
    t = pts - f[..., None] * ns[:, :, None, :]
    t2 = jnp.sum(t * t, axis=-1)

    z = (t + 0.5) * float(G) - 0.5
    ci = jnp.ceil(z).astype(jnp.int32)
    flat = ci[..., 0] * (G * G) + ci[..., 1] * G + ci[..., 2]
    flat = jnp.clip(flat, 0, V - 1)
    iv = flat + jnp.arange(B, dtype=jnp.int32)[:, None, None] * V
    jv = (iv >> 1).reshape(-1)
    parity = (iv & 1)[..., None]

    ones = jnp.ones((B, P, N, 1), f32)
    zeros = jnp.zeros((B, P, N, 3), f32)
    u8 = jnp.concatenate([-2.0 * t, t2[..., None], ones, zeros], axis=-1)
    zeros8 = jnp.zeros((B, P, N, 8), f32)
    u16 = jnp.where(parity == 1,
                    jnp.concatenate([zeros8, u8], axis=-1),
                    jnp.concatenate([u8, zeros8], axis=-1))
    u16 = u16.reshape(-1)

    partial = _sc_loss(jv, u16, tab)
    return jnp.sum(partial) / (B * P)
